# baseline (speedup 1.0000x reference)
import functools

import jax
import jax.numpy as jnp
from jax import lax
from jax.experimental import pallas as pl
from jax.experimental.pallas import tpu as pltpu
from jax.experimental.pallas import tpu_sc as plsc

_K = 16
_N = 1024
_BIG = 3.0e38
_NW = 32
_CH = 64
_RI = 4


def _dist_body(x_ref, d2_ref):
    x = x_ref[0]
    xb = x.astype(jnp.bfloat16)
    g = lax.dot_general(xb, xb, (((0,), (0,)), ((), ())),
                        preferred_element_type=jnp.float32)
    sq_row = jnp.sum(x * x, axis=0, keepdims=True)
    sq_col = jnp.transpose(sq_row, (1, 0))
    d2_ref[0] = sq_col + sq_row - 2.0 * g


def _select_body(nrows, d2_hbm, out_hbm, rows_v, t16_v, cand_v):
    wid = lax.axis_index("s") * 2 + lax.axis_index("c")
    base = wid * (nrows // _NW)

    def chunk_body(ci, carry):
        rbase = base + ci * _CH
        pltpu.sync_copy(d2_hbm.at[pl.ds(rbase, _CH)], rows_v)

        def row_body(r4, carry2):
            r0 = r4 * _RI

            ms = [rows_v[r0 + k, pl.ds(0, 16)] for k in range(_RI)]
            for i in range(1, 64):
                for k in range(_RI):
                    ms[k] = jnp.minimum(ms[k], rows_v[r0 + k, pl.ds(16 * i, 16)])
            t_ubs = [lax.sort(ms[k])[15] for k in range(_RI)]

            cnts = [jnp.int32(0)] * _RI
            for i in range(64):
                for k in range(_RI):
                    v = rows_v[r0 + k, pl.ds(16 * i, 16)]
                    msk = v <= t_ubs[k]
                    plsc.store_compressed(
                        cand_v.at[k, pl.ds(cnts[k], 16)], v, mask=msk)
                    cnts[k] = cnts[k] + plsc.all_reduce_population_count(msk)[0]
            big = jnp.full((16,), _BIG, jnp.float32)
            for k in range(_RI):
                cand_v[k, pl.ds(cnts[k], 16)] = big

            for k in range(_RI):
                def merge(j, top):
                    vs = lax.sort(cand_v[k, pl.ds(16 * j, 16)])
                    return lax.sort(jnp.minimum(top, lax.rev(vs, (0,))))

                top = lax.fori_loop(0, (cnts[k] + 15) // 16, merge, big)
                t16_v[r0 + k] = top
            return carry2

        lax.fori_loop(0, _CH // _RI, row_body, 0)
        pltpu.sync_copy(t16_v, out_hbm.at[pl.ds(rbase, _CH)])
        return carry

    lax.fori_loop(0, nrows // _NW // _CH, chunk_body, 0)


def _sc_select(d2_flat):
    nrows = d2_flat.shape[0]
    return pl.kernel(
        functools.partial(_select_body, nrows),
        out_type=jax.ShapeDtypeStruct((nrows, 16), jnp.float32),
        mesh=plsc.VectorSubcoreMesh(core_axis_name="c", subcore_axis_name="s"),
        scratch_types=[pltpu.VMEM((_CH, _N), jnp.float32),
                       pltpu.VMEM((_CH, 16), jnp.float32),
                       pltpu.VMEM((_RI, _N + 16), jnp.float32)],
        compiler_params=pltpu.CompilerParams(needs_layout_passes=False),
    )(d2_flat)


def _lap_body(d2_ref, t_ref, out_ref):
    d2 = d2_ref[0]
    t_row = t_ref[0]
    t_col = jnp.transpose(t_row, (1, 0))
    w = jnp.exp(-0.5 * d2)
    mk = ((d2 <= t_row).astype(jnp.float32)
          + (d2 <= t_col).astype(jnp.float32))
    a = 0.5 * w * mk
    deg = jnp.maximum(jnp.sum(a, axis=0, keepdims=True), 1e-6)
    r_row = lax.rsqrt(deg)
    r_col = jnp.transpose(r_row, (1, 0))
    lap = -(r_col * a * r_row)
    eye = (lax.broadcasted_iota(jnp.int32, (_N, _N), 0)
           == lax.broadcasted_iota(jnp.int32, (_N, _N), 1))
    out_ref[0] = jnp.where(eye, 1.0 + lap, lap)


def kernel(xyz):
    b = xyz.shape[0]
    d2 = pl.pallas_call(
        _dist_body,
        grid=(b,),
        in_specs=[pl.BlockSpec((1, 3, _N), lambda i: (i, 0, 0))],
        out_specs=pl.BlockSpec((1, _N, _N), lambda i: (i, 0, 0)),
        out_shape=jax.ShapeDtypeStruct((b, _N, _N), jnp.float32),
    )(xyz)

    t16 = _sc_select(d2.reshape(b * _N, _N))
    thr = t16[:, _K - 1].reshape(b, 1, _N)

    return pl.pallas_call(
        _lap_body,
        grid=(b,),
        in_specs=[pl.BlockSpec((1, _N, _N), lambda i: (i, 0, 0)),
                  pl.BlockSpec((1, 1, _N), lambda i: (i, 0, 0))],
        out_specs=pl.BlockSpec((1, _N, _N), lambda i: (i, 0, 0)),
        out_shape=jax.ShapeDtypeStruct((b, _N, _N), jnp.float32),
    )(d2, thr)


# speedup vs baseline: 2.7933x; 2.7933x over previous
import functools

import jax
import jax.numpy as jnp
from jax import lax
from jax.experimental import pallas as pl
from jax.experimental.pallas import tpu as pltpu
from jax.experimental.pallas import tpu_sc as plsc

_K = 16
_N = 1024
_BIG = 3.0e38
_NW = 32
_CH = 64
_SB = 2


def _dist_body(x_ref, d2_ref):
    x = x_ref[0]
    xb = x.astype(jnp.bfloat16)
    g = lax.dot_general(xb, xb, (((0,), (0,)), ((), ())),
                        preferred_element_type=jnp.float32)
    sq_row = jnp.sum(x * x, axis=0, keepdims=True)
    sq_col = jnp.transpose(sq_row, (1, 0))
    d2_ref[0] = sq_col + sq_row - 2.0 * g


def _select_body(nrows, d2_hbm, out_hbm, rows_v, t16_v, cand_v):
    wid = lax.axis_index("s") * 2 + lax.axis_index("c")
    base = wid * (nrows // _NW)

    def chunk_body(ci, carry):
        rbase = base + ci * _CH
        pltpu.sync_copy(d2_hbm.at[pl.ds(rbase, _CH)], rows_v)

        def row_body(r, carry2):
            m = rows_v[r, pl.ds(0, 16)]
            for i in range(1, 64):
                m = jnp.minimum(m, rows_v[r, pl.ds(16 * i, 16)])
            t_ub = lax.sort(m)[15]

            cnt = jnp.int32(0)
            for i in range(64):
                v = rows_v[r, pl.ds(16 * i, 16)]
                msk = v <= t_ub
                plsc.store_compressed(cand_v.at[pl.ds(cnt, 16)], v, mask=msk)
                cnt = cnt + plsc.all_reduce_population_count(msk)[0]
            cand_v[pl.ds(cnt, 16)] = jnp.full((16,), _BIG, jnp.float32)

            def merge(j, top):
                vs = lax.sort(cand_v[pl.ds(16 * j, 16)])
                return lax.sort(jnp.minimum(top, lax.rev(vs, (0,))))

            top = lax.fori_loop(0, (cnt + 15) // 16, merge,
                                jnp.full((16,), _BIG, jnp.float32))
            t16_v[r] = top
            return carry2

        lax.fori_loop(0, _CH, row_body, 0)
        pltpu.sync_copy(t16_v, out_hbm.at[pl.ds(rbase, _CH)])
        return carry

    lax.fori_loop(0, nrows // _NW // _CH, chunk_body, 0)


def _sc_select(d2_flat):
    nrows = d2_flat.shape[0]
    return pl.kernel(
        functools.partial(_select_body, nrows),
        out_type=jax.ShapeDtypeStruct((nrows, 16), jnp.float32),
        mesh=plsc.VectorSubcoreMesh(core_axis_name="c", subcore_axis_name="s"),
        scratch_types=[pltpu.VMEM((_CH, _N), jnp.float32),
                       pltpu.VMEM((_CH, 16), jnp.float32),
                       pltpu.VMEM((_N + 16,), jnp.float32)],
        compiler_params=pltpu.CompilerParams(needs_layout_passes=False),
    )(d2_flat)


def _fused_body(x_ref, out_ref):
    x = x_ref[0]
    xb = x.astype(jnp.bfloat16)
    g = lax.dot_general(xb, xb, (((0,), (0,)), ((), ())),
                        preferred_element_type=jnp.float32)
    eye = (lax.broadcasted_iota(jnp.int32, (_N, _N), 0)
           == lax.broadcasted_iota(jnp.int32, (_N, _N), 1))
    sq_row = jnp.sum(x * x, axis=0, keepdims=True)
    sq_col = jnp.transpose(sq_row, (1, 0))
    d2 = sq_col + sq_row - 2.0 * g
    t_row = jnp.full((1, _N), -_BIG, dtype=jnp.float32)
    for _ in range(_K):
        t_row = jnp.min(jnp.where(d2 > t_row, d2, _BIG), axis=0, keepdims=True)
    t_col = jnp.transpose(t_row, (1, 0))
    w = jnp.exp(-0.5 * d2)
    mk = ((d2 <= t_row).astype(jnp.float32)
          + (d2 <= t_col).astype(jnp.float32))
    a = 0.5 * w * mk
    deg = jnp.maximum(jnp.sum(a, axis=0, keepdims=True), 1e-6)
    r_row = lax.rsqrt(deg)
    r_col = jnp.transpose(r_row, (1, 0))
    lap = -(r_col * a * r_row)
    out_ref[0] = jnp.where(eye, 1.0 + lap, lap)


def _lap_body(d2_ref, t_ref, out_ref):
    d2 = d2_ref[0]
    t_row = t_ref[0]
    t_col = jnp.transpose(t_row, (1, 0))
    w = jnp.exp(-0.5 * d2)
    mk = ((d2 <= t_row).astype(jnp.float32)
          + (d2 <= t_col).astype(jnp.float32))
    a = 0.5 * w * mk
    deg = jnp.maximum(jnp.sum(a, axis=0, keepdims=True), 1e-6)
    r_row = lax.rsqrt(deg)
    r_col = jnp.transpose(r_row, (1, 0))
    lap = -(r_col * a * r_row)
    eye = (lax.broadcasted_iota(jnp.int32, (_N, _N), 0)
           == lax.broadcasted_iota(jnp.int32, (_N, _N), 1))
    out_ref[0] = jnp.where(eye, 1.0 + lap, lap)


def kernel(xyz):
    b = xyz.shape[0]
    s = _SB if b > _SB else b
    xa, xb = xyz[:s], xyz[s:]

    d2a = pl.pallas_call(
        _dist_body,
        grid=(s,),
        in_specs=[pl.BlockSpec((1, 3, _N), lambda i: (i, 0, 0))],
        out_specs=pl.BlockSpec((1, _N, _N), lambda i: (i, 0, 0)),
        out_shape=jax.ShapeDtypeStruct((s, _N, _N), jnp.float32),
    )(xa)
    t16 = _sc_select(d2a.reshape(s * _N, _N))
    thr = t16[:, _K - 1].reshape(s, 1, _N)

    outb = pl.pallas_call(
        _fused_body,
        grid=(b - s,),
        in_specs=[pl.BlockSpec((1, 3, _N), lambda i: (i, 0, 0))],
        out_specs=pl.BlockSpec((1, _N, _N), lambda i: (i, 0, 0)),
        out_shape=jax.ShapeDtypeStruct((b - s, _N, _N), jnp.float32),
    )(xb) if b > s else None

    outa = pl.pallas_call(
        _lap_body,
        grid=(s,),
        in_specs=[pl.BlockSpec((1, _N, _N), lambda i: (i, 0, 0)),
                  pl.BlockSpec((1, 1, _N), lambda i: (i, 0, 0))],
        out_specs=pl.BlockSpec((1, _N, _N), lambda i: (i, 0, 0)),
        out_shape=jax.ShapeDtypeStruct((s, _N, _N), jnp.float32),
    )(d2a, thr)

    if outb is None:
        return outa
    return jnp.concatenate([outa, outb], axis=0)


# speedup vs baseline: 3.4449x; 1.2333x over previous
import functools

import jax
import jax.numpy as jnp
from jax import lax
from jax.experimental import pallas as pl
from jax.experimental.pallas import tpu as pltpu
from jax.experimental.pallas import tpu_sc as plsc

_K = 16
_N = 1024
_BIG = 3.0e38
_NW = 32
_CH = 64
_SB = 2


def _dist_body(x_ref, d2_ref):
    x = x_ref[0]
    xb = x.astype(jnp.bfloat16)
    g = lax.dot_general(xb, xb, (((0,), (0,)), ((), ())),
                        preferred_element_type=jnp.float32)
    sq_row = jnp.sum(x * x, axis=0, keepdims=True)
    sq_col = jnp.transpose(sq_row, (1, 0))
    d2_ref[0] = sq_col + sq_row - 2.0 * g


def _select_body(nrows, d2_hbm, out_hbm, rows_v, t16_v, cand_v):
    wid = lax.axis_index("s") * 2 + lax.axis_index("c")
    base = wid * (nrows // _NW)

    def chunk_body(ci, carry):
        rbase = base + ci * _CH
        pltpu.sync_copy(d2_hbm.at[pl.ds(rbase, _CH)], rows_v)

        def row_body(r, carry2):
            m = rows_v[r, pl.ds(0, 16)]
            for i in range(1, 64):
                m = jnp.minimum(m, rows_v[r, pl.ds(16 * i, 16)])
            t_ub = lax.sort(m)[15]

            cnt = jnp.int32(0)
            for i in range(64):
                v = rows_v[r, pl.ds(16 * i, 16)]
                msk = v <= t_ub
                plsc.store_compressed(cand_v.at[pl.ds(cnt, 16)], v, mask=msk)
                cnt = cnt + plsc.all_reduce_population_count(msk)[0]
            cand_v[pl.ds(cnt, 16)] = jnp.full((16,), _BIG, jnp.float32)

            def merge(j, top):
                vs = lax.sort(cand_v[pl.ds(16 * j, 16)])
                return lax.sort(jnp.minimum(top, lax.rev(vs, (0,))))

            top = lax.fori_loop(0, (cnt + 15) // 16, merge,
                                jnp.full((16,), _BIG, jnp.float32))
            t16_v[r] = top
            return carry2

        lax.fori_loop(0, _CH, row_body, 0)
        pltpu.sync_copy(t16_v, out_hbm.at[pl.ds(rbase, _CH)])
        return carry

    lax.fori_loop(0, nrows // _NW // _CH, chunk_body, 0)


def _sc_select(d2_flat):
    nrows = d2_flat.shape[0]
    return pl.kernel(
        functools.partial(_select_body, nrows),
        out_type=jax.ShapeDtypeStruct((nrows, 16), jnp.float32),
        mesh=plsc.VectorSubcoreMesh(core_axis_name="c", subcore_axis_name="s"),
        scratch_types=[pltpu.VMEM((_CH, _N), jnp.float32),
                       pltpu.VMEM((_CH, 16), jnp.float32),
                       pltpu.VMEM((_N + 16,), jnp.float32)],
        compiler_params=pltpu.CompilerParams(needs_layout_passes=False),
    )(d2_flat)


def _fused_body(x_ref, out_ref):
    x = x_ref[0]
    xb = x.astype(jnp.bfloat16)
    g = lax.dot_general(xb, xb, (((0,), (0,)), ((), ())),
                        preferred_element_type=jnp.float32)
    eye = (lax.broadcasted_iota(jnp.int32, (_N, _N), 0)
           == lax.broadcasted_iota(jnp.int32, (_N, _N), 1))
    sq_row = jnp.sum(x * x, axis=0, keepdims=True)
    sq_col = jnp.transpose(sq_row, (1, 0))
    d2 = sq_col + sq_row - 2.0 * g
    t_row = jnp.full((1, _N), -_BIG, dtype=jnp.float32)
    for _ in range(_K):
        t_row = jnp.min(jnp.where(d2 > t_row, d2, _BIG), axis=0, keepdims=True)
    t_col = jnp.transpose(t_row, (1, 0))
    w = jnp.exp(-0.5 * d2)
    mk = ((d2 <= t_row).astype(jnp.float32)
          + (d2 <= t_col).astype(jnp.float32))
    a = 0.5 * w * mk
    deg = jnp.maximum(jnp.sum(a, axis=0, keepdims=True), 1e-6)
    r_row = lax.rsqrt(deg)
    r_col = jnp.transpose(r_row, (1, 0))
    lap = -(r_col * a * r_row)
    out_ref[0] = jnp.where(eye, 1.0 + lap, lap)


def _lap_body(d2_ref, t_ref, _buf_ref, out_ref):
    d2 = d2_ref[0]
    t_row = t_ref[0]
    t_col = jnp.transpose(t_row, (1, 0))
    w = jnp.exp(-0.5 * d2)
    mk = ((d2 <= t_row).astype(jnp.float32)
          + (d2 <= t_col).astype(jnp.float32))
    a = 0.5 * w * mk
    deg = jnp.maximum(jnp.sum(a, axis=0, keepdims=True), 1e-6)
    r_row = lax.rsqrt(deg)
    r_col = jnp.transpose(r_row, (1, 0))
    lap = -(r_col * a * r_row)
    eye = (lax.broadcasted_iota(jnp.int32, (_N, _N), 0)
           == lax.broadcasted_iota(jnp.int32, (_N, _N), 1))
    out_ref[0] = jnp.where(eye, 1.0 + lap, lap)


def kernel(xyz):
    b = xyz.shape[0]
    s = _SB if b > _SB else b
    xa, xb = xyz[:s], xyz[s:]

    d2a = pl.pallas_call(
        _dist_body,
        grid=(s,),
        in_specs=[pl.BlockSpec((1, 3, _N), lambda i: (i, 0, 0))],
        out_specs=pl.BlockSpec((1, _N, _N), lambda i: (i, 0, 0)),
        out_shape=jax.ShapeDtypeStruct((s, _N, _N), jnp.float32),
    )(xa)
    t16 = _sc_select(d2a.reshape(s * _N, _N))
    thr = t16[:, _K - 1].reshape(s, 1, _N)

    outb = pl.pallas_call(
        _fused_body,
        grid=(b - s,),
        in_specs=[pl.BlockSpec((1, 3, _N), lambda i: (i, 0, 0))],
        out_specs=pl.BlockSpec((1, _N, _N), lambda i, s=s: (i + s, 0, 0)),
        out_shape=jax.ShapeDtypeStruct((b, _N, _N), jnp.float32),
    )(xb)

    return pl.pallas_call(
        _lap_body,
        grid=(s,),
        in_specs=[pl.BlockSpec((1, _N, _N), lambda i: (i, 0, 0)),
                  pl.BlockSpec((1, 1, _N), lambda i: (i, 0, 0)),
                  pl.BlockSpec(memory_space=pl.ANY)],
        out_specs=pl.BlockSpec((1, _N, _N), lambda i: (i, 0, 0)),
        out_shape=jax.ShapeDtypeStruct((b, _N, _N), jnp.float32),
        input_output_aliases={2: 0},
    )(d2a, thr, outb)
